# baseline (speedup 1.0000x reference)
import jax
import jax.numpy as jnp
from jax import lax
from jax.experimental import pallas as pl
from jax.experimental.pallas import tpu as pltpu
from jax.experimental.pallas import tpu_sc as plsc

_D = 64
_B_TOTAL = 16384 * 50
_NC = 2
_NS = 16
_NW = _NC * _NS
_BPW = _B_TOTAL // _NW
_G = 128
_GROUPS = _BPW // _G
_BT = _B_TOTAL // _G


def _body(table_hbm, ids_hbm, q_hbm, idx_v, buf0, buf1, blkT, sg0, sg1, sw0, sw1):
    wid = lax.axis_index("s") * _NC + lax.axis_index("c")
    grp_base = wid * _GROUPS

    pltpu.sync_copy(ids_hbm.at[pl.ds(grp_base, _GROUPS)], idx_v)

    bufs = (buf0, buf1)
    semg = (sg0, sg1)
    semw = (sw0, sw1)
    iota16 = jnp.arange(16, dtype=jnp.int32)
    fcols = [iota16 + (16 * m) for m in range(_D // 16)]

    def fire_gather(j, b):
        pltpu.async_copy(table_hbm.at[idx_v.at[j]], bufs[b], semg[b])

    def drain_gather(b):
        pltpu.make_async_copy(table_hbm.at[pl.ds(0, _G)], bufs[b], semg[b]).wait()

    def fire_writes(j, b):
        bt = grp_base + j
        for dt in range(8):
            pltpu.async_copy(blkT.at[b, pl.ds(8 * dt, 8), pl.ds(0, _G)],
                             q_hbm.at[dt, bt], semw[b])

    def drain_writes(b):
        pltpu.make_async_copy(
            q_hbm.at[pl.ds(0, 8), 0],
            blkT.at[b, pl.ds(0, 8), pl.ds(0, _G)], semw[b]).wait()

    def transpose_block(b):
        src = bufs[b]
        dst = blkT.at[b]

        @plsc.parallel_loop(0, _G, 1, unroll=32)
        def _(r):
            lane = jnp.full((16,), r, dtype=jnp.int32)
            for m in range(_D // 16):
                v = src[r, pl.ds(16 * m, 16)]
                plsc.store_scatter(dst, [fcols[m], lane], v)

    fire_gather(0, 0)

    def step(i, _):
        for b in (0, 1):
            j = 2 * i + b

            @pl.when(j + 1 < _GROUPS)
            def _():
                fire_gather(j + 1, 1 - b)

            drain_gather(b)

            @pl.when(i > 0)
            def _():
                drain_writes(b)

            transpose_block(b)
            fire_writes(j, b)
        return 0

    lax.fori_loop(0, _GROUPS // 2, step, 0)
    drain_writes(0)
    drain_writes(1)


@jax.jit
def _gather(table, ids2d):
    mesh = plsc.VectorSubcoreMesh(core_axis_name="c", subcore_axis_name="s")
    return pl.kernel(
        _body,
        mesh=mesh,
        compiler_params=pltpu.CompilerParams(
            use_tc_tiling_on_sc=False, needs_layout_passes=False),
        out_type=jax.ShapeDtypeStruct((8, _BT, 8, _G), jnp.float32),
        scratch_types=[
            pltpu.VMEM((_GROUPS, _G), jnp.int32),
            pltpu.VMEM((_G, _D), jnp.float32),
            pltpu.VMEM((_G, _D), jnp.float32),
            pltpu.VMEM((2, _D, _G + 1), jnp.float32),
            pltpu.SemaphoreType.DMA,
            pltpu.SemaphoreType.DMA,
            pltpu.SemaphoreType.DMA,
            pltpu.SemaphoreType.DMA,
        ],
    )(table, ids2d)


_V = 1000000
_VB = 16384


def _depad_body(in_ref, out_ref):
    y = in_ref[...].T
    y3 = y.reshape(_VB // 2, 2, _D)
    out_ref[:, 0:_D] = y3[:, 0, :]
    out_ref[:, _D:2 * _D] = y3[:, 1, :]


@jax.jit
def _to_linear(t_t):
    return pl.pallas_call(
        _depad_body,
        grid=(pl.cdiv(_V, _VB),),
        in_specs=[pl.BlockSpec((_D, _VB), lambda j: (0, j))],
        out_specs=pl.BlockSpec((_VB // 2, 2 * _D), lambda j: (j, 0)),
        out_shape=jax.ShapeDtypeStruct((_V // 2, 2 * _D), jnp.float32),
    )(t_t)


def kernel(input_ids, table):
    ids2d = input_ids.reshape(_NW * _GROUPS, _G).astype(jnp.int32)
    table_lin = _to_linear(table.T).reshape(_V, _D)
    q = _gather(table_lin, ids2d)
    return jnp.transpose(q, (1, 3, 0, 2)).reshape(_B_TOTAL, 1, _D)


# speedup vs baseline: 1.0301x; 1.0301x over previous
import jax
import jax.numpy as jnp
from jax import lax
from jax.experimental import pallas as pl
from jax.experimental.pallas import tpu as pltpu
from jax.experimental.pallas import tpu_sc as plsc

_D = 64
_B_TOTAL = 16384 * 50
_NC = 2
_NS = 16
_NW = _NC * _NS
_BPW = _B_TOTAL // _NW
_G = 128
_GROUPS = _BPW // _G
_BT = _B_TOTAL // _G


def _body(table_hbm, ids_hbm, q_hbm, idx_v, buf0, buf1, blkT, sg0, sg1, sw0, sw1):
    wid = lax.axis_index("s") * _NC + lax.axis_index("c")
    grp_base = wid * _GROUPS

    pltpu.sync_copy(ids_hbm.at[pl.ds(grp_base, _GROUPS)], idx_v)

    bufs = (buf0, buf1)
    semg = (sg0, sg1)
    semw = (sw0, sw1)
    iota16 = jnp.arange(16, dtype=jnp.int32)
    fcols = [iota16 + (16 * m) for m in range(_D // 16)]

    def fire_gather(j, b):
        pltpu.async_copy(table_hbm.at[idx_v.at[j]], bufs[b], semg[b])

    def drain_gather(b):
        pltpu.make_async_copy(table_hbm.at[pl.ds(0, _G)], bufs[b], semg[b]).wait()

    def fire_writes(j, b):
        bt = grp_base + j
        for dt in range(8):
            pltpu.async_copy(blkT.at[b, pl.ds(8 * dt, 8), pl.ds(0, _G)],
                             q_hbm.at[dt, bt], semw[b])

    def drain_writes(b):
        pltpu.make_async_copy(
            q_hbm.at[pl.ds(0, 8), 0],
            blkT.at[b, pl.ds(0, 8), pl.ds(0, _G)], semw[b]).wait()

    def transpose_block(b):
        src = bufs[b]
        dst = blkT.at[b]

        @plsc.parallel_loop(0, _G, 1, unroll=8)
        def _(r):
            lane = jnp.full((16,), r, dtype=jnp.int32)
            for m in range(_D // 16):
                v = src[r, pl.ds(16 * m, 16)]
                plsc.store_scatter(dst, [fcols[m], lane], v)

    fire_gather(0, 0)

    def step(i, _):
        for b in (0, 1):
            j = 2 * i + b

            @pl.when(j + 1 < _GROUPS)
            def _():
                fire_gather(j + 1, 1 - b)

            drain_gather(b)

            @pl.when(i > 0)
            def _():
                drain_writes(b)

            transpose_block(b)
            fire_writes(j, b)
        return 0

    lax.fori_loop(0, _GROUPS // 2, step, 0)
    drain_writes(0)
    drain_writes(1)


@jax.jit
def _gather(table, ids2d):
    mesh = plsc.VectorSubcoreMesh(core_axis_name="c", subcore_axis_name="s")
    return pl.kernel(
        _body,
        mesh=mesh,
        compiler_params=pltpu.CompilerParams(
            use_tc_tiling_on_sc=False, needs_layout_passes=False),
        out_type=jax.ShapeDtypeStruct((8, _BT, 8, _G), jnp.float32),
        scratch_types=[
            pltpu.VMEM((_GROUPS, _G), jnp.int32),
            pltpu.VMEM((_G, _D), jnp.float32),
            pltpu.VMEM((_G, _D), jnp.float32),
            pltpu.VMEM((2, _D, _G + 1), jnp.float32),
            pltpu.SemaphoreType.DMA,
            pltpu.SemaphoreType.DMA,
            pltpu.SemaphoreType.DMA,
            pltpu.SemaphoreType.DMA,
        ],
    )(table, ids2d)


_V = 1000000
_VB = 16384


def _depad_body(in_ref, out_ref):
    y = in_ref[...].T
    y3 = y.reshape(_VB // 2, 2, _D)
    out_ref[:, 0:_D] = y3[:, 0, :]
    out_ref[:, _D:2 * _D] = y3[:, 1, :]


@jax.jit
def _to_linear(t_t):
    return pl.pallas_call(
        _depad_body,
        grid=(pl.cdiv(_V, _VB),),
        in_specs=[pl.BlockSpec((_D, _VB), lambda j: (0, j))],
        out_specs=pl.BlockSpec((_VB // 2, 2 * _D), lambda j: (j, 0)),
        out_shape=jax.ShapeDtypeStruct((_V // 2, 2 * _D), jnp.float32),
    )(t_t)


def kernel(input_ids, table):
    ids2d = input_ids.reshape(_NW * _GROUPS, _G).astype(jnp.int32)
    table_lin = _to_linear(table.T).reshape(_V, _D)
    q = _gather(table_lin, ids2d)
    return jnp.transpose(q, (1, 3, 0, 2)).reshape(_B_TOTAL, 1, _D)
